# baseline (speedup 1.0000x reference)
import jax
import jax.numpy as jnp
from jax import lax
from jax.experimental import pallas as pl
from jax.experimental.pallas import tpu as pltpu
from jax.experimental.pallas import tpu_sc as plsc

TASKS = 1_000_000
DIM = 16
BATCH = 16384
CLIP = 1.0

_info = plsc.get_sparse_core_info()
_NC, _NS, _L = _info.num_cores, _info.num_subcores, _info.num_lanes
_NW = _NC * _NS
_BPW = BATCH // _NW
_NCHUNK = _BPW // 16


_HALF = _BPW // 2


def _sc_body(tbl_hbm, idx_hbm, out_hbm, idx_v, q_v, big_v, out_v, sem):
    wid = lax.axis_index("s") * _NC + lax.axis_index("c")
    base = wid * _BPW
    pltpu.sync_copy(idx_hbm.at[pl.ds(base, _BPW)], idx_v)

    iota = lax.iota(jnp.int32, 16)

    def half(h, _):
        hb = pl.multiple_of(h * _HALF, _HALF)

        def qbody(c, _):
            o = pl.multiple_of(c * 16, 16)
            q_v[pl.ds(o, 16)] = lax.shift_right_logical(
                idx_v[pl.ds(hb + o, 16)], 3
            )
            return 0

        lax.fori_loop(0, _HALF // 16, qbody, 0)

        pltpu.async_copy(tbl_hbm.at[q_v], big_v, sem).wait()

        def ebody(c, _):
            o = pl.multiple_of(c * 16, 16)
            rows = iota + o
            sub = (idx_v[pl.ds(hb + o, 16)] & 7) * DIM
            for d in range(DIM):
                vals = plsc.load_gather(big_v, [rows, sub + d])
                vals = jnp.minimum(jnp.maximum(vals, -CLIP), CLIP)
                plsc.store_scatter(out_v, [rows + hb, iota * 0 + d], vals)
            return 0

        lax.fori_loop(0, _HALF // 16, ebody, 0)
        return 0

    lax.fori_loop(0, 2, half, 0)

    pltpu.sync_copy(out_v, out_hbm.at[pl.ds(base, _BPW)])


@jax.jit
def _gather_clip(table, idx_flat):
    tbl128 = table.reshape(TASKS // 8, 128)
    mesh = plsc.VectorSubcoreMesh(core_axis_name="c", subcore_axis_name="s")
    kfn = pl.kernel(
        _sc_body,
        mesh=mesh,
        out_type=jax.ShapeDtypeStruct((BATCH, DIM), jnp.float32),
        scratch_types=[
            pltpu.VMEM((_BPW,), jnp.int32),
            pltpu.VMEM((_HALF,), jnp.int32),
            pltpu.VMEM((_HALF, 128), jnp.float32),
            pltpu.VMEM((_BPW, DIM), jnp.float32),
            pltpu.SemaphoreType.DMA,
        ],
        compiler_params=pltpu.CompilerParams(
            needs_layout_passes=False, use_tc_tiling_on_sc=True
        ),
    )
    return kfn(tbl128, idx_flat)


def kernel(idx, context_hat):
    return _gather_clip(context_hat, idx[..., 0])


# speedup vs baseline: 1.6902x; 1.6902x over previous
import jax
import jax.numpy as jnp
from jax import lax
from jax.experimental import pallas as pl
from jax.experimental.pallas import tpu as pltpu
from jax.experimental.pallas import tpu_sc as plsc

TASKS = 1_000_000
DIM = 16
BATCH = 16384
CLIP = 1.0

_info = plsc.get_sparse_core_info()
_NC, _NS, _L = _info.num_cores, _info.num_subcores, _info.num_lanes
_NW = _NC * _NS
_BPW = BATCH // _NW


def _sc_body(tbl_hbm, idx_hbm, out_hbm, idx_v, rows_v, sem):
    wid = lax.axis_index("s") * _NC + lax.axis_index("c")
    base = wid * _BPW
    pltpu.sync_copy(idx_hbm.at[pl.ds(base, _BPW)], idx_v)

    @plsc.parallel_loop(0, _BPW, step=16, unroll=2)
    def _issue(o):
        v = idx_v[pl.ds(o, 16)]
        for j in range(16):
            pltpu.async_copy(tbl_hbm.at[v[j]], rows_v.at[o + j], sem)

    @plsc.parallel_loop(0, _BPW, step=1, unroll=8)
    def _drain(i):
        pltpu.make_async_copy(tbl_hbm.at[0], rows_v.at[0], sem).wait()

    @plsc.parallel_loop(0, _BPW, step=1, unroll=8)
    def _clip(i):
        rows_v[i] = jnp.minimum(jnp.maximum(rows_v[i], -CLIP), CLIP)

    pltpu.sync_copy(rows_v, out_hbm.at[pl.ds(base, _BPW)])


@jax.jit
def _gather_clip(table, idx_flat):
    mesh = plsc.VectorSubcoreMesh(core_axis_name="c", subcore_axis_name="s")
    kfn = pl.kernel(
        _sc_body,
        mesh=mesh,
        out_type=jax.ShapeDtypeStruct((BATCH, DIM), jnp.float32),
        scratch_types=[
            pltpu.VMEM((_BPW,), jnp.int32),
            pltpu.VMEM((_BPW, DIM), jnp.float32),
            pltpu.SemaphoreType.DMA,
        ],
        compiler_params=pltpu.CompilerParams(use_tc_tiling_on_sc=True),
    )
    return kfn(table, idx_flat)


def kernel(idx, context_hat):
    return _gather_clip(context_hat, idx[..., 0])
